# baseline (speedup 1.0000x reference)
import functools

import jax
import jax.numpy as jnp
from jax import lax
from jax.experimental import pallas as pl
from jax.experimental.pallas import tpu as pltpu
from jax.experimental.pallas import tpu_sc as plsc

CHUNK = 128
NBUF = 5
LEAD = 2


@functools.lru_cache(maxsize=None)
def _make_gather(H, Bt, V, D):
    info = plsc.get_sparse_core_info()
    NC, NS = info.num_cores, info.num_subcores
    NW = NC * NS
    assert Bt % (NW * CHUNK) == 0 and H % NBUF == 0
    mesh = plsc.VectorSubcoreMesh(core_axis_name="c", subcore_axis_name="s")

    @functools.partial(
        pl.kernel,
        mesh=mesh,
        out_type=jax.ShapeDtypeStruct((H * Bt, D), jnp.float32),
        scratch_types=[
            pltpu.VMEM((H, CHUNK), jnp.int32),
            pltpu.VMEM((NBUF, CHUNK, D), jnp.float32),
            [pltpu.SemaphoreType.DMA] * NBUF,
            [pltpu.SemaphoreType.DMA] * NBUF,
        ],
    )
    def gather_kernel(idx_hbm, table_hbm, out_hbm, idx_v, rows_v, gsems, osems):
        wid = lax.axis_index("s") * NC + lax.axis_index("c")
        col = wid * CHUNK
        pltpu.sync_copy(idx_hbm.at[:, pl.ds(col, CHUNK)], idx_v)

        for j in range(LEAD):
            pltpu.async_copy(table_hbm.at[idx_v.at[j]], rows_v.at[j], gsems[j])

        def group(g, _):
            j0 = g * NBUF
            for b in range(NBUF):
                j = j0 + b
                jn = j + LEAD
                sn = (b + LEAD) % NBUF
                jw = jn - NBUF

                @pl.when((jw >= 0) & (jn < H))
                def _():
                    pltpu.make_async_copy(
                        rows_v.at[sn],
                        out_hbm.at[pl.ds(jw * Bt + col, CHUNK)],
                        osems[sn],
                    ).wait()

                @pl.when(jn < H)
                def _():
                    pltpu.async_copy(
                        table_hbm.at[idx_v.at[jn]], rows_v.at[sn], gsems[sn]
                    )

                pltpu.make_async_copy(
                    table_hbm.at[idx_v.at[j]], rows_v.at[b], gsems[b]
                ).wait()
                pltpu.async_copy(
                    rows_v.at[b], out_hbm.at[pl.ds(j * Bt + col, CHUNK)], osems[b]
                )

            return 0

        lax.fori_loop(0, H // NBUF, group, 0)

        for b in range(NBUF):
            j = H - NBUF + b
            pltpu.make_async_copy(
                rows_v.at[j % NBUF],
                out_hbm.at[pl.ds(j * Bt + col, CHUNK)],
                osems[j % NBUF],
            ).wait()

    return gather_kernel


def kernel(x, table):
    B_, H_ = x.shape
    V, D = table.shape
    x_t = x.T.astype(jnp.int32)
    out = _make_gather(H_, B_, V, D)(x_t, table)
    return out.reshape(H_, B_, D).transpose(1, 0, 2)


# speedup vs baseline: 1.0009x; 1.0009x over previous
import functools

import jax
import jax.numpy as jnp
from jax import lax
from jax.experimental import pallas as pl
from jax.experimental.pallas import tpu as pltpu
from jax.experimental.pallas import tpu_sc as plsc

CHUNK = 128
NBUF = 5
LEAD = 3


@functools.lru_cache(maxsize=None)
def _make_gather(H, Bt, V, D):
    info = plsc.get_sparse_core_info()
    NC, NS = info.num_cores, info.num_subcores
    NW = NC * NS
    assert Bt % (NW * CHUNK) == 0 and H % NBUF == 0
    mesh = plsc.VectorSubcoreMesh(core_axis_name="c", subcore_axis_name="s")

    @functools.partial(
        pl.kernel,
        mesh=mesh,
        out_type=jax.ShapeDtypeStruct((H * Bt, D), jnp.float32),
        scratch_types=[
            pltpu.VMEM((H, CHUNK), jnp.int32),
            pltpu.VMEM((NBUF, CHUNK, D), jnp.float32),
            [pltpu.SemaphoreType.DMA] * NBUF,
            [pltpu.SemaphoreType.DMA] * NBUF,
        ],
    )
    def gather_kernel(idx_hbm, table_hbm, out_hbm, idx_v, rows_v, gsems, osems):
        wid = lax.axis_index("s") * NC + lax.axis_index("c")
        col = wid * CHUNK
        pltpu.sync_copy(idx_hbm.at[:, pl.ds(col, CHUNK)], idx_v)

        for j in range(LEAD):
            pltpu.async_copy(table_hbm.at[idx_v.at[j]], rows_v.at[j], gsems[j])

        def group(g, _):
            j0 = g * NBUF
            for b in range(NBUF):
                j = j0 + b
                jn = j + LEAD
                sn = (b + LEAD) % NBUF
                jw = jn - NBUF

                @pl.when((jw >= 0) & (jn < H))
                def _():
                    pltpu.make_async_copy(
                        rows_v.at[sn],
                        out_hbm.at[pl.ds(jw * Bt + col, CHUNK)],
                        osems[sn],
                    ).wait()

                @pl.when(jn < H)
                def _():
                    pltpu.async_copy(
                        table_hbm.at[idx_v.at[jn]], rows_v.at[sn], gsems[sn]
                    )

                pltpu.make_async_copy(
                    table_hbm.at[idx_v.at[j]], rows_v.at[b], gsems[b]
                ).wait()
                pltpu.async_copy(
                    rows_v.at[b], out_hbm.at[pl.ds(j * Bt + col, CHUNK)], osems[b]
                )

            return 0

        lax.fori_loop(0, H // NBUF, group, 0)

        for b in range(NBUF):
            j = H - NBUF + b
            pltpu.make_async_copy(
                rows_v.at[j % NBUF],
                out_hbm.at[pl.ds(j * Bt + col, CHUNK)],
                osems[j % NBUF],
            ).wait()

    return gather_kernel


def kernel(x, table):
    B_, H_ = x.shape
    V, D = table.shape
    x_t = x.T.astype(jnp.int32)
    out = _make_gather(H_, B_, V, D)(x_t, table)
    return out.reshape(H_, B_, D).transpose(1, 0, 2)


# speedup vs baseline: 1.0051x; 1.0042x over previous
import functools

import jax
import jax.numpy as jnp
from jax import lax
from jax.experimental import pallas as pl
from jax.experimental.pallas import tpu as pltpu
from jax.experimental.pallas import tpu_sc as plsc

CHUNK = 128
NBUF = 5
LEAD = 4


@functools.lru_cache(maxsize=None)
def _make_gather(H, Bt, V, D):
    info = plsc.get_sparse_core_info()
    NC, NS = info.num_cores, info.num_subcores
    NW = NC * NS
    assert Bt % (NW * CHUNK) == 0 and H % NBUF == 0
    mesh = plsc.VectorSubcoreMesh(core_axis_name="c", subcore_axis_name="s")

    @functools.partial(
        pl.kernel,
        mesh=mesh,
        out_type=jax.ShapeDtypeStruct((H * Bt, D), jnp.float32),
        scratch_types=[
            pltpu.VMEM((H, CHUNK), jnp.int32),
            pltpu.VMEM((NBUF, CHUNK, D), jnp.float32),
            [pltpu.SemaphoreType.DMA] * NBUF,
            [pltpu.SemaphoreType.DMA] * NBUF,
        ],
    )
    def gather_kernel(idx_hbm, table_hbm, out_hbm, idx_v, rows_v, gsems, osems):
        wid = lax.axis_index("s") * NC + lax.axis_index("c")
        col = wid * CHUNK
        pltpu.sync_copy(idx_hbm.at[:, pl.ds(col, CHUNK)], idx_v)

        for j in range(LEAD):
            pltpu.async_copy(table_hbm.at[idx_v.at[j]], rows_v.at[j], gsems[j])

        def group(g, _):
            j0 = g * NBUF
            for b in range(NBUF):
                j = j0 + b
                jn = j + LEAD
                sn = (b + LEAD) % NBUF
                jw = jn - NBUF

                @pl.when((jw >= 0) & (jn < H))
                def _():
                    pltpu.make_async_copy(
                        rows_v.at[sn],
                        out_hbm.at[pl.ds(jw * Bt + col, CHUNK)],
                        osems[sn],
                    ).wait()

                @pl.when(jn < H)
                def _():
                    pltpu.async_copy(
                        table_hbm.at[idx_v.at[jn]], rows_v.at[sn], gsems[sn]
                    )

                pltpu.make_async_copy(
                    table_hbm.at[idx_v.at[j]], rows_v.at[b], gsems[b]
                ).wait()
                pltpu.async_copy(
                    rows_v.at[b], out_hbm.at[pl.ds(j * Bt + col, CHUNK)], osems[b]
                )

            return 0

        lax.fori_loop(0, H // NBUF, group, 0)

        for b in range(NBUF):
            j = H - NBUF + b
            pltpu.make_async_copy(
                rows_v.at[j % NBUF],
                out_hbm.at[pl.ds(j * Bt + col, CHUNK)],
                osems[j % NBUF],
            ).wait()

    return gather_kernel


def kernel(x, table):
    B_, H_ = x.shape
    V, D = table.shape
    x_t = x.T.astype(jnp.int32)
    out = _make_gather(H_, B_, V, D)(x_t, table)
    return out.reshape(H_, B_, D).transpose(1, 0, 2)


# speedup vs baseline: 1.0053x; 1.0002x over previous
import functools

import jax
import jax.numpy as jnp
from jax import lax
from jax.experimental import pallas as pl
from jax.experimental.pallas import tpu as pltpu
from jax.experimental.pallas import tpu_sc as plsc

CHUNK = 128
NBUF = 5
LEAD = 4


@functools.lru_cache(maxsize=None)
def _make_gather(H, Bt, V, D):
    info = plsc.get_sparse_core_info()
    NC, NS = info.num_cores, info.num_subcores
    NW = NC * NS
    assert Bt % (NW * CHUNK) == 0 and H % NBUF == 0
    mesh = plsc.VectorSubcoreMesh(core_axis_name="c", subcore_axis_name="s")

    @functools.partial(
        pl.kernel,
        mesh=mesh,
        out_type=jax.ShapeDtypeStruct((H * Bt, D), jnp.float32),
        scratch_types=[
            pltpu.VMEM((H, CHUNK), jnp.int32),
            pltpu.VMEM((NBUF, CHUNK, D), jnp.float32),
            [pltpu.SemaphoreType.DMA] * NBUF,
            [pltpu.SemaphoreType.DMA] * NBUF,
            pltpu.SemaphoreType.DMA,
        ],
    )
    def gather_kernel(
        idx_hbm, table_hbm, out_hbm, idx_v, rows_v, gsems, osems, isem
    ):
        wid = lax.axis_index("s") * NC + lax.axis_index("c")
        col = wid * CHUNK
        pltpu.sync_copy(
            idx_hbm.at[pl.ds(0, 8), pl.ds(col, CHUNK)], idx_v.at[pl.ds(0, 8)]
        )
        rest = pltpu.async_copy(
            idx_hbm.at[pl.ds(8, H - 8), pl.ds(col, CHUNK)],
            idx_v.at[pl.ds(8, H - 8)],
            isem,
        )

        for j in range(LEAD):
            pltpu.async_copy(table_hbm.at[idx_v.at[j]], rows_v.at[j], gsems[j])
        rest.wait()

        def group(g, _):
            j0 = g * NBUF
            for b in range(NBUF):
                j = j0 + b
                jn = j + LEAD
                sn = (b + LEAD) % NBUF
                jw = jn - NBUF

                @pl.when((jw >= 0) & (jn < H))
                def _():
                    pltpu.make_async_copy(
                        rows_v.at[sn],
                        out_hbm.at[pl.ds(jw * Bt + col, CHUNK)],
                        osems[sn],
                    ).wait()

                @pl.when(jn < H)
                def _():
                    pltpu.async_copy(
                        table_hbm.at[idx_v.at[jn]], rows_v.at[sn], gsems[sn]
                    )

                pltpu.make_async_copy(
                    table_hbm.at[idx_v.at[j]], rows_v.at[b], gsems[b]
                ).wait()
                pltpu.async_copy(
                    rows_v.at[b], out_hbm.at[pl.ds(j * Bt + col, CHUNK)], osems[b]
                )

            return 0

        lax.fori_loop(0, H // NBUF, group, 0)

        for b in range(NBUF):
            j = H - NBUF + b
            pltpu.make_async_copy(
                rows_v.at[j % NBUF],
                out_hbm.at[pl.ds(j * Bt + col, CHUNK)],
                osems[j % NBUF],
            ).wait()

    return gather_kernel


def kernel(x, table):
    B_, H_ = x.shape
    V, D = table.shape
    x_t = x.T.astype(jnp.int32)
    out = _make_gather(H_, B_, V, D)(x_t, table)
    return out.reshape(H_, B_, D).transpose(1, 0, 2)
